# Initial kernel scaffold; baseline (speedup 1.0000x reference)
import jax
import jax.numpy as jnp
from jax.experimental import pallas as pl


def kernel(x, edge_spatial, edge_attr, alpha, s0_ws, s0_bs, s0_wn, s0_bn, s1_ws, s1_bs, s1_wn, s1_bn, a0_ws, a0_bs, a0_wn, a0_bn, a1_ws, a1_bs, a1_wn, a1_bn):
    raise NotImplementedError("write your pallas kernel here")


# speedup vs baseline: 5.5480x; 5.5480x over previous
import functools

import jax
import jax.numpy as jnp
from jax import lax
from jax.experimental import pallas as pl
from jax.experimental.pallas import tpu as pltpu
from jax.experimental.pallas import tpu_sc as plsc

N = 10000
D = 128
E = 320000
NC = 2
NS = 16
NW = NC * NS
K = 80
EPT = E // NW
NCHUNK = EPT // K
STRIPE = 632
LAST = N - (NS - 1) * STRIPE
BN = 1000
GN = N // BN

assert NCHUNK % 2 == 1 and EPT % K == 0 and K % 16 == 0 and K <= 128
assert STRIPE % 8 == 0 and LAST % 8 == 0 and 0 < LAST <= STRIPE

ZB = 64


def _dot(a, b):
    return lax.dot_general(a, b, (((1,), (0,)), ((), ())),
                           precision=lax.Precision.HIGHEST,
                           preferred_element_type=jnp.float32)


def _fill(ref, rows, value):
    vec = jnp.full((16,), value, jnp.float32)

    @pl.loop(0, rows)
    def _body(r):
        for q in range(D // 16):
            ref[r, pl.ds(q * 16, 16)] = vec


def _zero_rows(zbuf, acc, r0, nrows):
    full, rem = divmod(nrows, ZB)
    for q in range(full):
        pltpu.sync_copy(zbuf, acc.at[pl.ds(r0 + q * ZB, ZB)])
    if rem:
        pltpu.sync_copy(zbuf.at[pl.ds(0, rem)], acc.at[pl.ds(r0 + full * ZB, rem)])


def _zero_stripe(s, zbuf, acc):
    r0 = s * STRIPE

    @pl.when(s < NS - 1)
    def _full():
        _zero_rows(zbuf, acc, r0, STRIPE)

    @pl.when(s == NS - 1)
    def _last():
        _zero_rows(zbuf, acc, r0, LAST)


def _copy_out(c, s, acc, out_hbm):
    r0 = s * STRIPE

    @pl.when(s < NS - 1)
    def _full():
        pltpu.sync_copy(acc.at[pl.ds(r0, STRIPE)],
                        out_hbm.at[pl.ds(c * N + r0, STRIPE)])

    @pl.when(s == NS - 1)
    def _last():
        pltpu.sync_copy(acc.at[pl.ds(r0, LAST)],
                        out_hbm.at[pl.ds(c * N + r0, LAST)])


@functools.cache
def _make_sc_segment_sum(y_rows):
    mesh = plsc.VectorSubcoreMesh(core_axis_name="c", subcore_axis_name="s")

    def body(y_hbm, row_hbm, col_hbm, out_hbm,
             cidx0, cidx1, ridx0, ridx1, rows0, rows1, zbuf, acc,
             gsem0, gsem1, isem0, isem1):
        c = lax.axis_index("c")
        s = lax.axis_index("s")
        base_e = (c * NS + s) * EPT

        _fill(zbuf, ZB, 0.0)
        _zero_stripe(s, zbuf, acc)
        plsc.subcore_barrier()

        cidx = (cidx0, cidx1)
        ridx = (ridx0, ridx1)
        rows = (rows0, rows1)
        gsem = (gsem0, gsem1)
        isem = (isem0, isem1)

        def load_idx(j, b):
            st = base_e + j * K
            dc = pltpu.async_copy(col_hbm.at[pl.ds(st, K)], cidx[b], isem[b])
            dr = pltpu.async_copy(row_hbm.at[pl.ds(st, K)], ridx[b], isem[b])
            return dc, dr

        def wait_idx(b):
            pltpu.make_async_copy(col_hbm.at[pl.ds(0, K)], cidx[b], isem[b]).wait()
            pltpu.make_async_copy(row_hbm.at[pl.ds(0, K)], ridx[b], isem[b]).wait()

        def start_gather(b):
            pltpu.async_copy(y_hbm.at[cidx[b]], rows[b], gsem[b])

        def wait_gather(b):
            pltpu.make_async_copy(y_hbm.at[cidx[b]], rows[b], gsem[b]).wait()

        dc, dr = load_idx(0, 0)
        dc.wait()
        dr.wait()
        start_gather(0)
        load_idx(1, 1)

        @pl.loop(0, NCHUNK - 1, step=2)
        def _chunks(jb):
            for b in (0, 1):
                j = jb + b
                nb = 1 - b
                wait_idx(nb)
                wait_gather(b)
                start_gather(nb)
                pltpu.sync_copy(rows[b], acc.at[ridx[b]], add=True)
                jn2 = jnp.minimum(j + 2, NCHUNK - 1)
                load_idx(jn2, b)

        wait_idx(1)
        wait_gather(0)
        pltpu.sync_copy(rows0, acc.at[ridx0], add=True)

        plsc.subcore_barrier()
        _copy_out(c, s, acc, out_hbm)

    return pl.kernel(
        body,
        out_type=jax.ShapeDtypeStruct((2 * N, D), jnp.float32),
        mesh=mesh,
        scratch_types=[
            pltpu.VMEM((K,), jnp.int32),
            pltpu.VMEM((K,), jnp.int32),
            pltpu.VMEM((K,), jnp.int32),
            pltpu.VMEM((K,), jnp.int32),
            pltpu.VMEM((K, D), jnp.float32),
            pltpu.VMEM((K, D), jnp.float32),
            pltpu.VMEM((ZB, D), jnp.float32),
            pltpu.VMEM_SHARED((N, D), jnp.float32),
            pltpu.SemaphoreType.DMA,
            pltpu.SemaphoreType.DMA,
            pltpu.SemaphoreType.DMA,
            pltpu.SemaphoreType.DMA,
        ],
    )


@functools.cache
def _make_sc_degree():
    mesh = plsc.VectorSubcoreMesh(core_axis_name="c", subcore_axis_name="s")

    def body(row_hbm, out_hbm, ridx0, ridx1, ones_buf, zbuf, acc,
             isem0, isem1):
        c = lax.axis_index("c")
        s = lax.axis_index("s")
        base_e = (c * NS + s) * EPT

        _fill(ones_buf, K, 1.0)
        _fill(zbuf, ZB, 0.0)
        _zero_stripe(s, zbuf, acc)
        plsc.subcore_barrier()

        ridx = (ridx0, ridx1)
        isem = (isem0, isem1)

        def load_idx(j, b):
            pltpu.async_copy(row_hbm.at[pl.ds(base_e + j * K, K)],
                             ridx[b], isem[b])

        def wait_idx(b):
            pltpu.make_async_copy(row_hbm.at[pl.ds(0, K)], ridx[b], isem[b]).wait()

        load_idx(0, 0)
        load_idx(1, 1)

        @pl.loop(0, NCHUNK - 1, step=2)
        def _chunks(jb):
            for b in (0, 1):
                j = jb + b
                wait_idx(b)
                pltpu.sync_copy(ones_buf, acc.at[ridx[b]], add=True)
                jn2 = jnp.minimum(j + 2, NCHUNK - 1)
                load_idx(jn2, b)

        wait_idx(0)
        pltpu.sync_copy(ones_buf, acc.at[ridx0], add=True)
        wait_idx(1)

        plsc.subcore_barrier()
        _copy_out(c, s, acc, out_hbm)

    return pl.kernel(
        body,
        out_type=jax.ShapeDtypeStruct((2 * N, D), jnp.float32),
        mesh=mesh,
        scratch_types=[
            pltpu.VMEM((K,), jnp.int32),
            pltpu.VMEM((K,), jnp.int32),
            pltpu.VMEM((K, D), jnp.float32),
            pltpu.VMEM((ZB, D), jnp.float32),
            pltpu.VMEM_SHARED((N, D), jnp.float32),
            pltpu.SemaphoreType.DMA,
            pltpu.SemaphoreType.DMA,
        ],
    )


def _tc1(x, wn0, ws0, b0, interpret=False):
    def body(x_ref, wn_ref, ws_ref, b_ref, y_ref, base_ref):
        xb = x_ref[...]
        y_ref[...] = _dot(xb, wn_ref[0])
        base_ref[...] = _dot(xb, ws_ref[0]) + b_ref[0]

    return pl.pallas_call(
        body,
        grid=(2, GN),
        in_specs=[
            pl.BlockSpec((BN, D), lambda g, i: (i, 0)),
            pl.BlockSpec((1, D, D), lambda g, i: (g, 0, 0)),
            pl.BlockSpec((1, D, D), lambda g, i: (g, 0, 0)),
            pl.BlockSpec((1, 1, D), lambda g, i: (g, 0, 0)),
        ],
        out_specs=[
            pl.BlockSpec((BN, D), lambda g, i: (g * GN + i, 0)),
            pl.BlockSpec((BN, D), lambda g, i: (g * GN + i, 0)),
        ],
        out_shape=[
            jax.ShapeDtypeStruct((2 * N, D), jnp.float32),
            jax.ShapeDtypeStruct((2 * N, D), jnp.float32),
        ],
        interpret=interpret,
    )(x, wn0, ws0, b0)


def _tc2(g):
    def body(s0a_ref, s0b_ref, ca_ref, cb_ref, base0_ref, wn_ref, ws_ref,
             b_ref, y_ref, base_ref):
        cnt = ca_ref[...] + cb_ref[...]
        nei = (s0a_ref[...] + s0b_ref[...]) / (cnt + 1e-12)
        h0 = jnp.maximum(base0_ref[...] + nei, 0.0)
        y_ref[...] = _dot(h0, wn_ref[...])
        base_ref[...] = _dot(h0, ws_ref[...]) + b_ref[...]

    def call(s0p, cntp, b0, wn1, ws1, b1, interpret=False):
        return pl.pallas_call(
            body,
            grid=(GN,),
            in_specs=[
                pl.BlockSpec((BN, D), lambda i: (i, 0)),
                pl.BlockSpec((BN, D), lambda i: (GN + i, 0)),
                pl.BlockSpec((BN, D), lambda i: (i, 0)),
                pl.BlockSpec((BN, D), lambda i: (GN + i, 0)),
                pl.BlockSpec((BN, D), lambda i: (g * GN + i, 0)),
                pl.BlockSpec((D, D), lambda i: (0, 0)),
                pl.BlockSpec((D, D), lambda i: (0, 0)),
                pl.BlockSpec((1, D), lambda i: (0, 0)),
            ],
            out_specs=[
                pl.BlockSpec((BN, D), lambda i: (i, 0)),
                pl.BlockSpec((BN, D), lambda i: (i, 0)),
            ],
            out_shape=[
                jax.ShapeDtypeStruct((N, D), jnp.float32),
                jax.ShapeDtypeStruct((N, D), jnp.float32),
            ],
            interpret=interpret,
        )(s0p, s0p, cntp, cntp, b0, wn1, ws1, b1)

    return call


def _tc3(b1s, b1a, s1ps, s1pa, cntps, cntpa, alpha, interpret=False):
    def body(b1s_ref, b1a_ref, ssa_ref, ssb_ref, saa_ref, sab_ref,
             csa_ref, csb_ref, caa_ref, cab_ref, a_ref, out_ref):
        wgt = 1.0 / (1.0 + jnp.exp(-a_ref[0, 0]))
        cs = csa_ref[...] + csb_ref[...] + 1e-12
        ca = caa_ref[...] + cab_ref[...] + 1e-12
        hs = jnp.maximum(b1s_ref[...] + (ssa_ref[...] + ssb_ref[...]) / cs,
                         0.0)
        ha = jnp.maximum(b1a_ref[...] + (saa_ref[...] + sab_ref[...]) / ca,
                         0.0)
        out_ref[...] = wgt * hs + (1.0 - wgt) * ha

    lo = lambda i: (i, 0)
    hi = lambda i: (GN + i, 0)
    return pl.pallas_call(
        body,
        grid=(GN,),
        in_specs=[
            pl.BlockSpec((BN, D), lo),
            pl.BlockSpec((BN, D), lo),
            pl.BlockSpec((BN, D), lo),
            pl.BlockSpec((BN, D), hi),
            pl.BlockSpec((BN, D), lo),
            pl.BlockSpec((BN, D), hi),
            pl.BlockSpec((BN, D), lo),
            pl.BlockSpec((BN, D), hi),
            pl.BlockSpec((BN, D), lo),
            pl.BlockSpec((BN, D), hi),
            pl.BlockSpec(memory_space=pltpu.SMEM),
        ],
        out_specs=pl.BlockSpec((BN, D), lo),
        out_shape=jax.ShapeDtypeStruct((N, D), jnp.float32),
        interpret=interpret,
    )(b1s, b1a, s1ps, s1ps, s1pa, s1pa, cntps, cntps, cntpa, cntpa, alpha)


def kernel(x, edge_spatial, edge_attr, alpha,
           s0_ws, s0_bs, s0_wn, s0_bn, s1_ws, s1_bs, s1_wn, s1_bn,
           a0_ws, a0_bs, a0_wn, a0_bn, a1_ws, a1_bs, a1_wn, a1_bn):
    es = edge_spatial.astype(jnp.int32)
    ea = edge_attr.astype(jnp.int32)
    row_s, col_s = es[0], es[1]
    row_a = ea[0]
    col_a_g = ea[1] + N
    col_a_l = ea[1]

    wn0 = jnp.stack([s0_wn, a0_wn])
    ws0 = jnp.stack([s0_ws, a0_ws])
    b0 = jnp.stack([s0_bs + s0_bn, a0_bs + a0_bn])[:, None, :]
    b1_s = jnp.reshape(s1_bs + s1_bn, (1, D))
    b1_a = jnp.reshape(a1_bs + a1_bn, (1, D))
    alpha2 = jnp.reshape(alpha, (1, 1)).astype(jnp.float32)

    seg2n = _make_sc_segment_sum(2 * N)
    segn = _make_sc_segment_sum(N)
    deg = _make_sc_degree()

    cntp_s = deg(row_s)
    cntp_a = deg(row_a)
    y0, base0 = _tc1(x, wn0, ws0, b0)
    s0p_s = seg2n(y0, row_s, col_s)
    s0p_a = seg2n(y0, row_a, col_a_g)
    y1_s, base1_s = _tc2(0)(s0p_s, cntp_s, base0, s1_wn, s1_ws, b1_s)
    y1_a, base1_a = _tc2(1)(s0p_a, cntp_a, base0, a1_wn, a1_ws, b1_a)
    s1p_s = segn(y1_s, row_s, col_s)
    s1p_a = segn(y1_a, row_a, col_a_l)
    return _tc3(base1_s, base1_a, s1p_s, s1p_a, cntp_s, cntp_a, alpha2)


# speedup vs baseline: 5.6034x; 1.0100x over previous
import functools

import jax
import jax.numpy as jnp
from jax import lax
from jax.experimental import pallas as pl
from jax.experimental.pallas import tpu as pltpu
from jax.experimental.pallas import tpu_sc as plsc

N = 10000
D = 128
E = 320000
NC = 2
NS = 16
K = 80
EPT = E // NS
NCHUNK = EPT // K
STRIPE = 632
LAST = N - (NS - 1) * STRIPE
ZB = 64
SUMW = 640
SUMLAST = N - (NS - 1) * SUMW
NPAD = NS * SUMW
SUMB = 512
BN = 1000
GN = N // BN

assert NCHUNK % 2 == 0 and EPT % K == 0 and K % 8 == 0 and K <= 128
assert STRIPE % 8 == 0 and LAST % 8 == 0 and 0 < LAST <= STRIPE
assert SUMW % 128 == 0 and SUMB % 128 == 0 and SUMW % SUMB == SUMW - SUMB
assert SUMLAST % 8 == 0


def _dot(a, b):
    return lax.dot_general(a, b, (((1,), (0,)), ((), ())),
                           precision=lax.Precision.HIGHEST,
                           preferred_element_type=jnp.float32)


def _zero_stripe(s, zbuf, acc):
    def _zero_rows(r0, nrows):
        for q in range(nrows // ZB):
            pltpu.sync_copy(zbuf.at[pl.ds(0, ZB)],
                            acc.at[pl.ds(r0 + q * ZB, ZB)])
        rem = nrows % ZB
        if rem:
            pltpu.sync_copy(zbuf.at[pl.ds(0, rem)],
                            acc.at[pl.ds(r0 + (nrows // ZB) * ZB, rem)])

    r0 = s * STRIPE

    @pl.when(s < NS - 1)
    def _full():
        _zero_rows(r0, STRIPE)

    @pl.when(s == NS - 1)
    def _last():
        _zero_rows(r0, LAST)


def _copy_out(c, s, acc, out_hbm):
    r0 = s * STRIPE

    @pl.when(s < NS - 1)
    def _full():
        pltpu.sync_copy(acc.at[pl.ds(r0, STRIPE)],
                        out_hbm.at[pl.ds(c * N + r0, STRIPE)])

    @pl.when(s == NS - 1)
    def _last():
        pltpu.sync_copy(acc.at[pl.ds(r0, LAST)],
                        out_hbm.at[pl.ds(c * N + r0, LAST)])


@functools.cache
def _make_sc_segment_sum():
    mesh = plsc.VectorSubcoreMesh(core_axis_name="c", subcore_axis_name="s")

    def body(y_hbm, row_hbm, col_hbm, out_hbm,
             cidx0, cidx1, ridx0, ridx1, rows0, rows1, acc,
             gsem0, gsem1, isem0, isem1):
        c = lax.axis_index("c")
        s = lax.axis_index("s")
        base_e = c * E + s * EPT

        zvec = jnp.zeros((16,), jnp.float32)

        @pl.loop(0, ZB)
        def _zrow(r):
            for q in range(D // 16):
                rows0[r, pl.ds(q * 16, 16)] = zvec

        _zero_stripe(s, rows0, acc)
        plsc.subcore_barrier()

        cidx = (cidx0, cidx1)
        ridx = (ridx0, ridx1)
        rows = (rows0, rows1)
        gsem = (gsem0, gsem1)
        isem = (isem0, isem1)

        def load_idx(j, b):
            st = base_e + j * K
            dc = pltpu.async_copy(col_hbm.at[pl.ds(st, K)], cidx[b], isem[b])
            dr = pltpu.async_copy(row_hbm.at[pl.ds(st, K)], ridx[b], isem[b])
            return dc, dr

        def wait_idx(b):
            pltpu.make_async_copy(col_hbm.at[pl.ds(0, K)], cidx[b], isem[b]).wait()
            pltpu.make_async_copy(row_hbm.at[pl.ds(0, K)], ridx[b], isem[b]).wait()

        def start_gather(b):
            pltpu.async_copy(y_hbm.at[cidx[b]], rows[b], gsem[b])

        def wait_gather(b):
            pltpu.make_async_copy(y_hbm.at[cidx[b]], rows[b], gsem[b]).wait()

        dc, dr = load_idx(0, 0)
        dc.wait()
        dr.wait()
        start_gather(0)
        load_idx(1, 1)

        @pl.loop(0, NCHUNK, step=2)
        def _chunks(jb):
            for b in (0, 1):
                j = jb + b
                nb = 1 - b
                wait_idx(nb)
                wait_gather(b)
                start_gather(nb)
                pltpu.sync_copy(rows[b], acc.at[ridx[b]], add=True)
                jn2 = jnp.minimum(j + 2, NCHUNK - 1)
                load_idx(jn2, b)

        wait_gather(0)
        wait_idx(1)

        plsc.subcore_barrier()
        _copy_out(c, s, acc, out_hbm)

    return pl.kernel(
        body,
        out_type=jax.ShapeDtypeStruct((2 * N, D), jnp.float32),
        mesh=mesh,
        scratch_types=[
            pltpu.VMEM((K,), jnp.int32),
            pltpu.VMEM((K,), jnp.int32),
            pltpu.VMEM((K,), jnp.int32),
            pltpu.VMEM((K,), jnp.int32),
            pltpu.VMEM((K, D), jnp.float32),
            pltpu.VMEM((K, D), jnp.float32),
            pltpu.VMEM_SHARED((N, D), jnp.float32),
            pltpu.SemaphoreType.DMA,
            pltpu.SemaphoreType.DMA,
            pltpu.SemaphoreType.DMA,
            pltpu.SemaphoreType.DMA,
        ],
    )


@functools.cache
def _make_sc_degree():
    mesh = plsc.VectorSubcoreMesh(core_axis_name="c", subcore_axis_name="s")

    def body(row_hbm, out_hbm, ridx0, ridx1, ones_buf, acc, isem0, isem1):
        c = lax.axis_index("c")
        s = lax.axis_index("s")
        base_e = c * E + s * EPT

        zvec = jnp.zeros((16,), jnp.float32)

        @pl.loop(0, ZB)
        def _zrow(r):
            for q in range(D // 16):
                ones_buf[r, pl.ds(q * 16, 16)] = zvec

        _zero_stripe(s, ones_buf, acc)
        ovec = jnp.ones((16,), jnp.float32)

        @pl.loop(0, K)
        def _orow(r):
            for q in range(D // 16):
                ones_buf[r, pl.ds(q * 16, 16)] = ovec

        plsc.subcore_barrier()

        ridx = (ridx0, ridx1)
        isem = (isem0, isem1)

        def load_idx(j, b):
            pltpu.async_copy(row_hbm.at[pl.ds(base_e + j * K, K)],
                             ridx[b], isem[b])

        def wait_idx(b):
            pltpu.make_async_copy(row_hbm.at[pl.ds(0, K)], ridx[b], isem[b]).wait()

        load_idx(0, 0)
        load_idx(1, 1)

        @pl.loop(0, NCHUNK, step=2)
        def _chunks(jb):
            for b in (0, 1):
                j = jb + b
                wait_idx(b)
                pltpu.sync_copy(ones_buf, acc.at[ridx[b]], add=True)
                jn2 = jnp.minimum(j + 2, NCHUNK - 1)
                load_idx(jn2, b)

        wait_idx(0)
        wait_idx(1)

        plsc.subcore_barrier()
        _copy_out(c, s, acc, out_hbm)

    return pl.kernel(
        body,
        out_type=jax.ShapeDtypeStruct((2 * N, D), jnp.float32),
        mesh=mesh,
        scratch_types=[
            pltpu.VMEM((K,), jnp.int32),
            pltpu.VMEM((K,), jnp.int32),
            pltpu.VMEM((K, D), jnp.float32),
            pltpu.VMEM_SHARED((N, D), jnp.float32),
            pltpu.SemaphoreType.DMA,
            pltpu.SemaphoreType.DMA,
        ],
    )


def _tc1(x, wn0, ws0, b0, interpret=False):
    def body(x_ref, wn_ref, ws_ref, b_ref, y_ref, base_ref):
        xb = x_ref[...]
        y_ref[...] = _dot(xb, wn_ref[0])
        base_ref[...] = _dot(xb, ws_ref[0]) + b_ref[0]

    return pl.pallas_call(
        body,
        grid=(2, GN),
        in_specs=[
            pl.BlockSpec((BN, D), lambda g, i: (i, 0)),
            pl.BlockSpec((1, D, D), lambda g, i: (g, 0, 0)),
            pl.BlockSpec((1, D, D), lambda g, i: (g, 0, 0)),
            pl.BlockSpec((1, 1, D), lambda g, i: (g, 0, 0)),
        ],
        out_specs=[
            pl.BlockSpec((BN, D), lambda g, i: (g * GN + i, 0)),
            pl.BlockSpec((BN, D), lambda g, i: (g * GN + i, 0)),
        ],
        out_shape=[
            jax.ShapeDtypeStruct((2 * N, D), jnp.float32),
            jax.ShapeDtypeStruct((2 * N, D), jnp.float32),
        ],
        interpret=interpret,
    )(x, wn0, ws0, b0)


def _tc2(s0, cnt, base0, wn1, ws1, b1, interpret=False):
    def body(s0_ref, cnt_ref, base0_ref, wn_ref, ws_ref, b_ref,
             y_ref, base_ref):
        nei = s0_ref[...] / (cnt_ref[...] + 1e-12)
        h0 = jnp.maximum(base0_ref[...] + nei, 0.0)
        y_ref[...] = _dot(h0, wn_ref[0])
        base_ref[...] = _dot(h0, ws_ref[0]) + b_ref[0]

    return pl.pallas_call(
        body,
        grid=(2, GN),
        in_specs=[
            pl.BlockSpec((BN, D), lambda g, i: (g * GN + i, 0)),
            pl.BlockSpec((BN, D), lambda g, i: (g * GN + i, 0)),
            pl.BlockSpec((BN, D), lambda g, i: (g * GN + i, 0)),
            pl.BlockSpec((1, D, D), lambda g, i: (g, 0, 0)),
            pl.BlockSpec((1, D, D), lambda g, i: (g, 0, 0)),
            pl.BlockSpec((1, 1, D), lambda g, i: (g, 0, 0)),
        ],
        out_specs=[
            pl.BlockSpec((BN, D), lambda g, i: (g * GN + i, 0)),
            pl.BlockSpec((BN, D), lambda g, i: (g * GN + i, 0)),
        ],
        out_shape=[
            jax.ShapeDtypeStruct((2 * N, D), jnp.float32),
            jax.ShapeDtypeStruct((2 * N, D), jnp.float32),
        ],
        interpret=interpret,
    )(s0, cnt, base0, wn1, ws1, b1)


def _tc3(base1, s1, cnt, alpha, interpret=False):
    def body(b1s_ref, b1a_ref, s1s_ref, s1a_ref, cs_ref, ca_ref, a_ref,
             out_ref):
        wgt = 1.0 / (1.0 + jnp.exp(-a_ref[0, 0]))
        hs = jnp.maximum(b1s_ref[...] + s1s_ref[...] / (cs_ref[...] + 1e-12),
                         0.0)
        ha = jnp.maximum(b1a_ref[...] + s1a_ref[...] / (ca_ref[...] + 1e-12),
                         0.0)
        out_ref[...] = wgt * hs + (1.0 - wgt) * ha

    lo = lambda i: (i, 0)
    hi = lambda i: (GN + i, 0)
    return pl.pallas_call(
        body,
        grid=(GN,),
        in_specs=[
            pl.BlockSpec((BN, D), lo),
            pl.BlockSpec((BN, D), hi),
            pl.BlockSpec((BN, D), lo),
            pl.BlockSpec((BN, D), hi),
            pl.BlockSpec((BN, D), lo),
            pl.BlockSpec((BN, D), hi),
            pl.BlockSpec(memory_space=pltpu.SMEM),
        ],
        out_specs=pl.BlockSpec((BN, D), lo),
        out_shape=jax.ShapeDtypeStruct((N, D), jnp.float32),
        interpret=interpret,
    )(base1, base1, s1, s1, cnt, cnt, alpha)


def kernel(x, edge_spatial, edge_attr, alpha,
           s0_ws, s0_bs, s0_wn, s0_bn, s1_ws, s1_bs, s1_wn, s1_bn,
           a0_ws, a0_bs, a0_wn, a0_bn, a1_ws, a1_bs, a1_wn, a1_bn):
    es = edge_spatial.astype(jnp.int32)
    ea = edge_attr.astype(jnp.int32)
    row_all = jnp.concatenate([es[0], ea[0]])
    col_all = jnp.concatenate([es[1], ea[1] + N])

    wn0 = jnp.stack([s0_wn, a0_wn])
    ws0 = jnp.stack([s0_ws, a0_ws])
    b0 = jnp.stack([s0_bs + s0_bn, a0_bs + a0_bn])[:, None, :]
    wn1 = jnp.stack([s1_wn, a1_wn])
    ws1 = jnp.stack([s1_ws, a1_ws])
    b1 = jnp.stack([s1_bs + s1_bn, a1_bs + a1_bn])[:, None, :]
    alpha2 = jnp.reshape(alpha, (1, 1)).astype(jnp.float32)

    cnt = _make_sc_degree()(row_all)
    y0, base0 = _tc1(x, wn0, ws0, b0)
    s0 = _make_sc_segment_sum()(y0, row_all, col_all)
    y1, base1 = _tc2(s0, cnt, base0, wn1, ws1, b1)
    s1 = _make_sc_segment_sum()(y1, row_all, col_all)
    return _tc3(base1, s1, cnt, alpha2)
